# Initial kernel scaffold; baseline (speedup 1.0000x reference)
import jax
import jax.numpy as jnp
from jax.experimental import pallas as pl


def kernel(indices, word_embeddings):
    raise NotImplementedError("write your pallas kernel here")


# speedup vs baseline: 1.7172x; 1.7172x over previous
import jax
import jax.numpy as jnp
from jax.experimental import pallas as pl
from jax.experimental.pallas import tpu as pltpu

_VOCAB = 10
_HIDDEN = 1024
_BLOCK = 2048


def _lookup_block(idx_ref, table_ref, out_ref):
    idx = idx_ref[0, 0, :]
    one_hot = (idx[:, None] == jax.lax.iota(jnp.int32, _VOCAB)[None, :])
    one_hot = one_hot.astype(jnp.float32)
    out_ref[0, :, :] = jnp.dot(
        one_hot, table_ref[...], preferred_element_type=jnp.float32
    )


def kernel(indices, word_embeddings):
    n_rows, n_cols = indices.shape
    total = n_rows * n_cols
    num_blocks = total // _BLOCK
    idx3 = indices.astype(jnp.int32).reshape(num_blocks, 1, _BLOCK)

    out = pl.pallas_call(
        _lookup_block,
        grid=(num_blocks,),
        in_specs=[
            pl.BlockSpec((1, 1, _BLOCK), lambda i: (i, 0, 0)),
            pl.BlockSpec((_VOCAB, _HIDDEN), lambda i: (0, 0)),
        ],
        out_specs=pl.BlockSpec((1, _BLOCK, _HIDDEN), lambda i: (i, 0, 0)),
        out_shape=jax.ShapeDtypeStruct((num_blocks, _BLOCK, _HIDDEN), jnp.float32),
    )(idx3, word_embeddings)
    return out.reshape(n_rows, n_cols, _HIDDEN)
